# speedup vs baseline: 2.9674x; 2.9674x over previous
import functools

import jax
import jax.numpy as jnp
from jax import lax
from jax.experimental import pallas as pl
from jax.experimental.pallas import tpu as pltpu
from jax.experimental.pallas import tpu_sc as plsc

EMB_DIM = 128
NUM_CORES = 2
NUM_SUBCORES = 16
NUM_WORKERS = NUM_CORES * NUM_SUBCORES
CHUNK = 128


@functools.partial(jax.jit, static_argnames=("n_chunks",))
def _embedding_gather(token_ids_3d, table, *, n_chunks):
    b_per_w = n_chunks * CHUNK
    total = NUM_WORKERS * b_per_w
    mesh = plsc.VectorSubcoreMesh(core_axis_name="c", subcore_axis_name="s")

    @functools.partial(
        pl.kernel,
        mesh=mesh,
        out_type=jax.ShapeDtypeStruct((total, EMB_DIM), jnp.float32),
        scratch_types=[
            pltpu.VMEM((n_chunks, CHUNK), jnp.int32),
            pltpu.VMEM((CHUNK, EMB_DIM), jnp.float32),
            pltpu.SemaphoreType.DMA,
        ],
    )
    def k(idx_hbm, table_hbm, out_hbm, idx_v, rows_v, sem):
        wid = lax.axis_index("s") * NUM_CORES + lax.axis_index("c")
        base = wid * b_per_w
        pltpu.sync_copy(idx_hbm.at[wid], idx_v)

        def body(j, _):
            pltpu.async_copy(table_hbm.at[idx_v.at[j]], rows_v, sem).wait()
            pltpu.sync_copy(rows_v, out_hbm.at[pl.ds(base + j * CHUNK, CHUNK)])
            return 0

        lax.fori_loop(0, n_chunks, body, 0)

    return k(token_ids_3d, table)


def kernel(token_ids, embedding_map):
    orig_shape = token_ids.shape
    flat = token_ids.reshape(-1).astype(jnp.int32)
    n = flat.shape[0]
    assert n % (NUM_WORKERS * CHUNK) == 0
    n_chunks = n // (NUM_WORKERS * CHUNK)
    idx3 = flat.reshape(NUM_WORKERS, n_chunks, CHUNK)
    out = _embedding_gather(idx3, embedding_map, n_chunks=n_chunks)
    return out.reshape(*orig_shape, EMB_DIM)


# speedup vs baseline: 3.1241x; 1.0528x over previous
import functools

import jax
import jax.numpy as jnp
from jax import lax
from jax.experimental import pallas as pl
from jax.experimental.pallas import tpu as pltpu
from jax.experimental.pallas import tpu_sc as plsc

EMB_DIM = 128
NUM_CORES = 2
NUM_SUBCORES = 16
NUM_WORKERS = NUM_CORES * NUM_SUBCORES
CHUNK = 128


@functools.partial(jax.jit, static_argnames=("n_chunks",))
def _embedding_gather(token_ids_3d, table, *, n_chunks):
    b_per_w = n_chunks * CHUNK
    total = NUM_WORKERS * b_per_w
    mesh = plsc.VectorSubcoreMesh(core_axis_name="c", subcore_axis_name="s")

    assert n_chunks % 2 == 0
    n_pairs = n_chunks // 2

    @functools.partial(
        pl.kernel,
        mesh=mesh,
        out_type=jax.ShapeDtypeStruct((total, EMB_DIM), jnp.float32),
        scratch_types=[
            pltpu.VMEM((n_chunks, CHUNK), jnp.int32),
            pltpu.VMEM((CHUNK, EMB_DIM), jnp.float32),
            pltpu.VMEM((CHUNK, EMB_DIM), jnp.float32),
            pltpu.SemaphoreType.DMA,
            pltpu.SemaphoreType.DMA,
            pltpu.SemaphoreType.DMA,
            pltpu.SemaphoreType.DMA,
        ],
    )
    def k(idx_hbm, table_hbm, out_hbm, idx_v, buf0, buf1, g0, g1, s0, s1):
        wid = lax.axis_index("s") * NUM_CORES + lax.axis_index("c")
        base = wid * b_per_w
        pltpu.sync_copy(idx_hbm.at[wid], idx_v)

        bufs = (buf0, buf1)
        gsems = (g0, g1)
        ssems = (s0, s1)

        def out_at(c):
            return out_hbm.at[pl.ds(base + c * CHUNK, CHUNK)]

        pltpu.async_copy(table_hbm.at[idx_v.at[0]], buf0, g0)

        def body(p, _):
            for b in range(2):
                c = 2 * p + b
                other = 1 - b
                pltpu.make_async_copy(
                    table_hbm.at[idx_v.at[c]], bufs[b], gsems[b]
                ).wait()
                if b == 1:
                    pltpu.make_async_copy(bufs[other], out_at(c), ssems[other]).wait()
                else:
                    @pl.when(p > 0)
                    def _():
                        pltpu.make_async_copy(
                            bufs[other], out_at(c), ssems[other]
                        ).wait()
                if b == 0:
                    pltpu.async_copy(
                        table_hbm.at[idx_v.at[c + 1]], bufs[other], gsems[other]
                    )
                else:
                    @pl.when(p < n_pairs - 1)
                    def _():
                        pltpu.async_copy(
                            table_hbm.at[idx_v.at[c + 1]], bufs[other], gsems[other]
                        )
                pltpu.async_copy(bufs[b], out_at(c), ssems[b])
            return 0

        lax.fori_loop(0, n_pairs, body, 0)
        pltpu.make_async_copy(buf1, out_at(n_chunks - 1), s1).wait()

    return k(token_ids_3d, table)


def kernel(token_ids, embedding_map):
    orig_shape = token_ids.shape
    flat = token_ids.reshape(-1).astype(jnp.int32)
    n = flat.shape[0]
    assert n % (NUM_WORKERS * CHUNK) == 0
    n_chunks = n // (NUM_WORKERS * CHUNK)
    idx3 = flat.reshape(NUM_WORKERS, n_chunks, CHUNK)
    out = _embedding_gather(idx3, embedding_map, n_chunks=n_chunks)
    return out.reshape(*orig_shape, EMB_DIM)


# speedup vs baseline: 5.6265x; 1.8010x over previous
import functools

import jax
import jax.numpy as jnp
from jax import lax
from jax.experimental import pallas as pl
from jax.experimental.pallas import tpu as pltpu
from jax.experimental.pallas import tpu_sc as plsc

EMB_DIM = 128
NUM_CORES = 2
NUM_SUBCORES = 16
NUM_WORKERS = NUM_CORES * NUM_SUBCORES
T_BUF = 8


@functools.partial(jax.jit, static_argnames=("n_tok", "seq"))
def _embedding_gather(token_ids_3d, table, *, n_tok, seq):
    tok_per_w = n_tok // NUM_WORKERS
    n_grp = tok_per_w // T_BUF
    mesh = plsc.VectorSubcoreMesh(core_axis_name="c", subcore_axis_name="s")

    @functools.partial(
        pl.kernel,
        mesh=mesh,
        out_type=jax.ShapeDtypeStruct((n_tok, seq, EMB_DIM), jnp.float32),
        scratch_types=[
            pltpu.VMEM((tok_per_w, seq), jnp.int32),
            pltpu.VMEM((T_BUF, seq, EMB_DIM), jnp.float32),
            pltpu.SemaphoreType.DMA,
        ],
    )
    def k(idx_hbm, table_hbm, out_hbm, idx_v, buf, gsem):
        wid = lax.axis_index("s") * NUM_CORES + lax.axis_index("c")
        tok0 = wid * tok_per_w
        pltpu.sync_copy(idx_hbm.at[wid], idx_v)

        def body(g, _):
            t0 = g * T_BUF
            for t in range(T_BUF):
                pltpu.async_copy(
                    table_hbm.at[idx_v.at[t0 + t]], buf.at[t], gsem
                )
            for t in range(T_BUF):
                pltpu.make_async_copy(
                    table_hbm.at[idx_v.at[t0 + t]], buf.at[t], gsem
                ).wait()
            pltpu.sync_copy(buf, out_hbm.at[pl.ds(tok0 + t0, T_BUF)])
            return 0

        lax.fori_loop(0, n_grp, body, 0)

    return k(token_ids_3d, table)


def kernel(token_ids, embedding_map):
    n_tok, seq = token_ids.shape
    assert n_tok % (NUM_WORKERS * T_BUF) == 0
    idx3 = token_ids.astype(jnp.int32).reshape(NUM_WORKERS, n_tok // NUM_WORKERS, seq)
    return _embedding_gather(idx3, embedding_map, n_tok=n_tok, seq=seq)
